# initial kernel scaffold (unmeasured)
import jax
import jax.numpy as jnp
from jax import lax
from jax.experimental import pallas as pl
from jax.experimental.pallas import tpu as pltpu


def kernel(
    x,
):
    def body(*refs):
        pass

    out_shape = jax.ShapeDtypeStruct(..., jnp.float32)
    return pl.pallas_call(body, out_shape=out_shape)(...)


# baseline (device time: 30102 ns/iter reference)
import jax
import jax.numpy as jnp
from jax import lax
from jax.experimental import pallas as pl
from jax.experimental.pallas import tpu as pltpu

N_DEV = 8


def kernel(x):
    m_rows, n_cols = x.shape

    def body(x_ref, out_ref, stats_ref, send_sems, recv_sems):
        my = lax.axis_index("i")

        barrier_sem = pltpu.get_barrier_semaphore()
        for d in range(1, N_DEV):
            tgt = lax.rem(my + d, N_DEV)
            pl.semaphore_signal(
                barrier_sem, inc=1,
                device_id=(tgt,), device_id_type=pl.DeviceIdType.MESH,
            )
        pl.semaphore_wait(barrier_sem, N_DEV - 1)

        xv = x_ref[:, :]
        m = jnp.max(xv, axis=1)
        e = jnp.exp(xv - m[:, None])
        s = jnp.sum(e, axis=1)
        stats_ref[0, 0, :] = m
        stats_ref[0, 1, :] = s

        rdmas = []
        for d in range(1, N_DEV):
            tgt = lax.rem(my + d, N_DEV)
            slot = N_DEV - d
            rdma = pltpu.make_async_remote_copy(
                src_ref=stats_ref.at[0],
                dst_ref=stats_ref.at[slot],
                send_sem=send_sems.at[d],
                recv_sem=recv_sems.at[slot],
                device_id=(tgt,),
                device_id_type=pl.DeviceIdType.MESH,
            )
            rdma.start()
            rdmas.append(rdma)

        out_ref[:, :] = e.astype(out_ref.dtype)

        for r in rdmas:
            r.wait_send()
        for r in rdmas:
            r.wait_recv()

        all_stats = stats_ref[:, :, :]
        all_m = all_stats[:, 0, :]
        all_s = all_stats[:, 1, :]
        gmax = jnp.max(all_m, axis=0)
        gsum = jnp.sum(all_s * jnp.exp(all_m - gmax[None, :]), axis=0)
        scale = jnp.exp(m - gmax) / gsum
        out_ref[:, :] = (
            out_ref[:, :].astype(jnp.float32) * scale[:, None]
        ).astype(out_ref.dtype)

    return pl.pallas_call(
        body,
        out_shape=jax.ShapeDtypeStruct((m_rows, n_cols), jnp.bfloat16),
        in_specs=[pl.BlockSpec(memory_space=pltpu.VMEM)],
        out_specs=pl.BlockSpec(memory_space=pltpu.VMEM),
        scratch_shapes=[
            pltpu.VMEM((N_DEV, 2, m_rows), jnp.float32),
            pltpu.SemaphoreType.DMA((N_DEV,)),
            pltpu.SemaphoreType.DMA((N_DEV,)),
        ],
        compiler_params=pltpu.CompilerParams(
            collective_id=0, vmem_limit_bytes=100 * 1024 * 1024
        ),
    )(x)


# device time: 23737 ns/iter; 1.2681x vs baseline; 1.2681x over previous
import jax
import jax.numpy as jnp
from jax import lax
from jax.experimental import pallas as pl
from jax.experimental.pallas import tpu as pltpu

N_DEV = 8
N_CHUNKS = 8


def kernel(x):
    m_rows, n_cols = x.shape
    rows_c = m_rows // N_CHUNKS
    sub_c = rows_c // 128

    def body(x_hbm, out_hbm, xv, ev, stats_ref,
             in_sems, out_sems, send_sems, recv_sems):
        my = lax.axis_index("i")

        barrier_sem = pltpu.get_barrier_semaphore()
        for d in range(1, N_DEV):
            tgt = lax.rem(my + d, N_DEV)
            pl.semaphore_signal(
                barrier_sem, inc=1,
                device_id=(tgt,), device_id_type=pl.DeviceIdType.MESH,
            )

        in_dmas = []
        for c in range(N_CHUNKS):
            dma = pltpu.make_async_copy(
                x_hbm.at[pl.ds(c * rows_c, rows_c), :],
                xv.at[pl.ds(c * rows_c, rows_c), :],
                in_sems.at[c],
            )
            dma.start()
            in_dmas.append(dma)

        deferred = None
        for c in range(N_CHUNKS):
            in_dmas[c].wait()
            xc = xv[pl.ds(c * rows_c, rows_c), :]
            mc = jnp.max(xc, axis=1)
            ec = jnp.exp(xc - mc[:, None])
            sc = jnp.sum(ec, axis=1)
            if c == N_CHUNKS - 1:
                deferred = ec
            else:
                ev[pl.ds(c * rows_c, rows_c), :] = ec.astype(ev.dtype)
            stats_ref[0, 0, pl.ds(c * sub_c, sub_c)] = mc.reshape(sub_c, 128)
            stats_ref[0, 1, pl.ds(c * sub_c, sub_c)] = sc.reshape(sub_c, 128)

        pl.semaphore_wait(barrier_sem, N_DEV - 1)
        rdmas = []
        for d in range(1, N_DEV):
            tgt = lax.rem(my + d, N_DEV)
            slot = N_DEV - d
            rdma = pltpu.make_async_remote_copy(
                src_ref=stats_ref.at[0],
                dst_ref=stats_ref.at[slot],
                send_sem=send_sems.at[d],
                recv_sem=recv_sems.at[slot],
                device_id=(tgt,),
                device_id_type=pl.DeviceIdType.MESH,
            )
            rdma.start()
            rdmas.append(rdma)
        ev[pl.ds((N_CHUNKS - 1) * rows_c, rows_c), :] = (
            deferred.astype(ev.dtype)
        )
        for r in rdmas:
            r.wait_recv()

        all_stats = stats_ref[...]
        all_m = all_stats[:, 0]
        all_s = all_stats[:, 1]
        gmax = jnp.max(all_m, axis=0)
        gsum = jnp.sum(all_s * jnp.exp(all_m - gmax[None]), axis=0)
        scale2 = jnp.exp(all_stats[0, 0] - gmax) / gsum
        scale = scale2.reshape(m_rows).astype(ev.dtype)

        out_dmas = []
        for c in range(N_CHUNKS):
            sl = pl.ds(c * rows_c, rows_c)
            sc_c = scale[c * rows_c:(c + 1) * rows_c]
            ev[sl, :] = ev[sl, :] * sc_c[:, None]
            dma = pltpu.make_async_copy(
                ev.at[sl, :], out_hbm.at[sl, :], out_sems.at[c]
            )
            dma.start()
            out_dmas.append(dma)
        for r in rdmas:
            r.wait_send()
        for dma in out_dmas:
            dma.wait()

    return pl.pallas_call(
        body,
        out_shape=jax.ShapeDtypeStruct((m_rows, n_cols), jnp.bfloat16),
        in_specs=[pl.BlockSpec(memory_space=pl.ANY)],
        out_specs=pl.BlockSpec(memory_space=pl.ANY),
        scratch_shapes=[
            pltpu.VMEM((m_rows, n_cols), jnp.float32),
            pltpu.VMEM((m_rows, n_cols), jnp.bfloat16),
            pltpu.VMEM((N_DEV, 2, m_rows // 128, 128), jnp.float32),
            pltpu.SemaphoreType.DMA((N_CHUNKS,)),
            pltpu.SemaphoreType.DMA((N_CHUNKS,)),
            pltpu.SemaphoreType.DMA((N_DEV,)),
            pltpu.SemaphoreType.DMA((N_DEV,)),
        ],
        compiler_params=pltpu.CompilerParams(
            collective_id=0, vmem_limit_bytes=100 * 1024 * 1024
        ),
    )(x)
